# Initial kernel scaffold; baseline (speedup 1.0000x reference)
import jax
import jax.numpy as jnp
from jax.experimental import pallas as pl


def kernel(x, edge_index, params):
    raise NotImplementedError("write your pallas kernel here")


# speedup vs baseline: 4.1259x; 4.1259x over previous
import functools

import jax
import jax.numpy as jnp
import numpy as np
from jax import lax
from jax.experimental import pallas as pl
from jax.experimental.pallas import tpu as pltpu
from jax.experimental.pallas import tpu_sc as plsc

_NC = 2
_NS = 16
_CHUNK = 80


def _segment_sum_sc(h, src, dst):
    N, H = h.shape
    E = src.shape[0]
    nw = _NC * _NS
    epw = E // nw
    nchunk = epw // _CHUNK
    assert epw * nw == E and nchunk * _CHUNK == epw
    wrows = 200
    nwc = N // wrows
    assert nwc * wrows == N

    mesh = plsc.VectorSubcoreMesh(core_axis_name="c", subcore_axis_name="s")

    @functools.partial(
        pl.kernel,
        out_type=jax.ShapeDtypeStruct((_NC, N, H), jnp.float32),
        mesh=mesh,
        scratch_types=[
            pltpu.VMEM((_CHUNK,), jnp.int32),
            pltpu.VMEM((_CHUNK,), jnp.int32),
            pltpu.VMEM((_CHUNK, H), jnp.float32),
            pltpu.VMEM((wrows, H), jnp.float32),
            pltpu.VMEM_SHARED((N, H), jnp.float32),
            pltpu.SemaphoreType.DMA,
        ],
    )
    def seg(h_hbm, src_hbm, dst_hbm, out_hbm, sidx, didx, rows, zbuf, agg, gsem):
        cid = lax.axis_index("c")
        sid = lax.axis_index("s")
        wid = cid * _NS + sid

        zero16 = jnp.zeros((16,), jnp.float32)

        @pl.loop(0, wrows)
        def _zero_rows(i):
            for j in range(H // 16):
                zbuf[i, pl.ds(j * 16, 16)] = zero16

        for i in range((nwc + _NS - 1) // _NS):
            blk = sid + i * _NS

            @pl.when(blk < nwc)
            def _():
                pltpu.sync_copy(zbuf, agg.at[pl.ds(blk * wrows, wrows)])
        plsc.subcore_barrier()

        ebase = wid * epw

        @pl.loop(0, nchunk)
        def _edge_chunk(c):
            off = ebase + c * _CHUNK
            pltpu.sync_copy(src_hbm.at[pl.ds(off, _CHUNK)], sidx)
            pltpu.sync_copy(dst_hbm.at[pl.ds(off, _CHUNK)], didx)
            pltpu.async_copy(h_hbm.at[sidx], rows, gsem).wait()
            pltpu.sync_copy(rows, agg.at[didx], add=True)

        plsc.subcore_barrier()
        for i in range((nwc + _NS - 1) // _NS):
            blk = sid + i * _NS

            @pl.when(blk < nwc)
            def _():
                pltpu.sync_copy(agg.at[pl.ds(blk * wrows, wrows)],
                                out_hbm.at[cid, pl.ds(blk * wrows, wrows)])

    return seg(h, src, dst)


_BLK = 1000


def _input_proj(x, W, b, g, beta):
    N, D = x.shape
    H = W.shape[1]

    def body(x_ref, w_ref, b_ref, g_ref, bb_ref, o_ref):
        h = jnp.dot(x_ref[...], w_ref[...],
                    preferred_element_type=jnp.float32,
                    precision=lax.Precision.HIGHEST) + b_ref[...]
        h = jnp.maximum(h, 0.0)
        mu = jnp.mean(h, axis=1, keepdims=True)
        var = jnp.mean((h - mu) ** 2, axis=1, keepdims=True)
        o_ref[...] = (h - mu) * lax.rsqrt(var + 1e-5) * g_ref[...] + bb_ref[...]

    return pl.pallas_call(
        body,
        grid=(N // _BLK,),
        in_specs=[
            pl.BlockSpec((_BLK, D), lambda i: (i, 0)),
            pl.BlockSpec((D, H), lambda i: (0, 0)),
            pl.BlockSpec((1, H), lambda i: (0, 0)),
            pl.BlockSpec((1, H), lambda i: (0, 0)),
            pl.BlockSpec((1, H), lambda i: (0, 0)),
        ],
        out_specs=pl.BlockSpec((_BLK, H), lambda i: (i, 0)),
        out_shape=jax.ShapeDtypeStruct((N, H), jnp.float32),
    )(x, W, b.reshape(1, H), g.reshape(1, H), beta.reshape(1, H))


def _gin_mlp(h, a0, a1, W1, b1, W2, b2, bn_g, bn_b):
    N, H = h.shape
    bn_scale = (bn_g / np.sqrt(1.0 + 1e-5)).reshape(1, H)

    def body(h_ref, a0_ref, a1_ref, w1_ref, b1_ref, w2_ref, b2_ref,
             s_ref, bb_ref, o_ref):
        z = h_ref[...] + a0_ref[...] + a1_ref[...]
        z = jnp.dot(z, w1_ref[...], preferred_element_type=jnp.float32,
                    precision=lax.Precision.HIGHEST) + b1_ref[...]
        z = jnp.maximum(z, 0.0)
        z = jnp.dot(z, w2_ref[...], preferred_element_type=jnp.float32,
                    precision=lax.Precision.HIGHEST) + b2_ref[...]
        z = z * s_ref[...] + bb_ref[...]
        o_ref[...] = jnp.maximum(z, 0.0)

    return pl.pallas_call(
        body,
        grid=(N // _BLK,),
        in_specs=[
            pl.BlockSpec((_BLK, H), lambda i: (i, 0)),
            pl.BlockSpec((_BLK, H), lambda i: (i, 0)),
            pl.BlockSpec((_BLK, H), lambda i: (i, 0)),
            pl.BlockSpec((H, H), lambda i: (0, 0)),
            pl.BlockSpec((1, H), lambda i: (0, 0)),
            pl.BlockSpec((H, H), lambda i: (0, 0)),
            pl.BlockSpec((1, H), lambda i: (0, 0)),
            pl.BlockSpec((1, H), lambda i: (0, 0)),
            pl.BlockSpec((1, H), lambda i: (0, 0)),
        ],
        out_specs=pl.BlockSpec((_BLK, H), lambda i: (i, 0)),
        out_shape=jax.ShapeDtypeStruct((N, H), jnp.float32),
    )(h, a0, a1, W1, b1.reshape(1, H), W2, b2.reshape(1, H),
      bn_scale, bn_b.reshape(1, H))


def _pool(h, Wp, bp):
    N, H = h.shape

    def body(h_ref, wp_ref, bp_ref, o_ref):
        hm = jnp.mean(h_ref[...], axis=0, keepdims=True)
        hx = jnp.max(h_ref[...], axis=0, keepdims=True)
        hc = jnp.concatenate([hm, hx], axis=1)
        o = jnp.dot(hc, wp_ref[...], preferred_element_type=jnp.float32,
                    precision=lax.Precision.HIGHEST) + bp_ref[...]
        o_ref[...] = jnp.maximum(o, 0.0)

    return pl.pallas_call(
        body,
        out_shape=jax.ShapeDtypeStruct((1, H), jnp.float32),
    )(h, Wp, bp.reshape(1, H))


def kernel(x, edge_index, params):
    src = edge_index[0]
    dst = edge_index[1]
    h = _input_proj(x, params["W_in"], params["b_in"],
                    params["ln_g"], params["ln_b"])
    for lp in params["layers"]:
        agg2 = _segment_sum_sc(h, src, dst)
        h = _gin_mlp(h, agg2[0], agg2[1], lp["W1"], lp["b1"],
                     lp["W2"], lp["b2"], lp["bn_g"], lp["bn_b"])
    return _pool(h, params["W_pool"], params["b_pool"])


# speedup vs baseline: 6.3614x; 1.5418x over previous
import functools

import jax
import jax.numpy as jnp
import numpy as np
from jax import lax
from jax.experimental import pallas as pl
from jax.experimental.pallas import tpu as pltpu
from jax.experimental.pallas import tpu_sc as plsc

_NC = 2
_NS = 16
_CHUNK = 80


def _segment_sum_sc(h, src, dst):
    N, H = h.shape
    E = src.shape[0]
    nw = _NC * _NS
    epw = E // nw
    nchunk = epw // _CHUNK
    assert epw * nw == E and nchunk * _CHUNK == epw
    wrows = 200
    nwc = N // wrows
    assert nwc * wrows == N

    mesh = plsc.VectorSubcoreMesh(core_axis_name="c", subcore_axis_name="s")

    @functools.partial(
        pl.kernel,
        out_type=jax.ShapeDtypeStruct((_NC, N, H), jnp.float32),
        mesh=mesh,
        scratch_types=[
            pltpu.VMEM((_CHUNK,), jnp.int32),
            pltpu.VMEM((_CHUNK,), jnp.int32),
            pltpu.VMEM((_CHUNK,), jnp.int32),
            pltpu.VMEM((_CHUNK,), jnp.int32),
            pltpu.VMEM((_CHUNK, H), jnp.float32),
            pltpu.VMEM((_CHUNK, H), jnp.float32),
            pltpu.VMEM((wrows, H), jnp.float32),
            pltpu.VMEM_SHARED((N, H), jnp.float32),
            pltpu.SemaphoreType.DMA,
            pltpu.SemaphoreType.DMA,
        ],
    )
    def seg(h_hbm, src_hbm, dst_hbm, out_hbm,
            sidx0, sidx1, didx0, didx1, rows0, rows1, zbuf, agg, sem0, sem1):
        cid = lax.axis_index("c")
        sid = lax.axis_index("s")
        wid = cid * _NS + sid
        ebase = wid * epw

        zero16 = jnp.zeros((16,), jnp.float32)

        @pl.loop(0, wrows)
        def _zero_rows(i):
            for j in range(H // 16):
                zbuf[i, pl.ds(j * 16, 16)] = zero16

        for i in range((nwc + _NS - 1) // _NS):
            blk = sid + i * _NS

            @pl.when(blk < nwc)
            def _():
                pltpu.sync_copy(zbuf, agg.at[pl.ds(blk * wrows, wrows)])
        plsc.subcore_barrier()

        def fire(c, sidx, didx, rows_ref, sem):
            off = ebase + c * _CHUNK
            pltpu.sync_copy(src_hbm.at[pl.ds(off, _CHUNK)], sidx)
            pltpu.sync_copy(dst_hbm.at[pl.ds(off, _CHUNK)], didx)
            pltpu.async_copy(h_hbm.at[sidx], rows_ref, sem)

        def drain_scatter(didx, rows_ref, sem):
            pltpu.make_async_copy(h_hbm.at[pl.ds(0, _CHUNK)], rows_ref,
                                  sem).wait()
            pltpu.sync_copy(rows_ref, agg.at[didx], add=True)

        fire(0, sidx0, didx0, rows0, sem0)

        @pl.loop(0, nchunk - 1, step=2)
        def _edge_pair(c):
            fire(c + 1, sidx1, didx1, rows1, sem1)
            drain_scatter(didx0, rows0, sem0)
            fire(c + 2, sidx0, didx0, rows0, sem0)
            drain_scatter(didx1, rows1, sem1)

        drain_scatter(didx0, rows0, sem0)

        plsc.subcore_barrier()
        for i in range((nwc + _NS - 1) // _NS):
            blk = sid + i * _NS

            @pl.when(blk < nwc)
            def _():
                pltpu.sync_copy(agg.at[pl.ds(blk * wrows, wrows)],
                                out_hbm.at[cid, pl.ds(blk * wrows, wrows)])

    return seg(h, src, dst)


_BLK = 1000


def _input_proj(x, W, b, g, beta):
    N, D = x.shape
    H = W.shape[1]

    def body(x_ref, w_ref, b_ref, g_ref, bb_ref, o_ref):
        h = jnp.dot(x_ref[...], w_ref[...],
                    preferred_element_type=jnp.float32,
                    precision=lax.Precision.HIGHEST) + b_ref[...]
        h = jnp.maximum(h, 0.0)
        mu = jnp.mean(h, axis=1, keepdims=True)
        var = jnp.mean((h - mu) ** 2, axis=1, keepdims=True)
        o_ref[...] = (h - mu) * lax.rsqrt(var + 1e-5) * g_ref[...] + bb_ref[...]

    return pl.pallas_call(
        body,
        grid=(N // _BLK,),
        in_specs=[
            pl.BlockSpec((_BLK, D), lambda i: (i, 0)),
            pl.BlockSpec((D, H), lambda i: (0, 0)),
            pl.BlockSpec((1, H), lambda i: (0, 0)),
            pl.BlockSpec((1, H), lambda i: (0, 0)),
            pl.BlockSpec((1, H), lambda i: (0, 0)),
        ],
        out_specs=pl.BlockSpec((_BLK, H), lambda i: (i, 0)),
        out_shape=jax.ShapeDtypeStruct((N, H), jnp.float32),
    )(x, W, b.reshape(1, H), g.reshape(1, H), beta.reshape(1, H))


def _gin_mlp(h, a0, a1, W1, b1, W2, b2, bn_g, bn_b):
    N, H = h.shape
    bn_scale = (bn_g / np.sqrt(1.0 + 1e-5)).reshape(1, H)

    def body(h_ref, a0_ref, a1_ref, w1_ref, b1_ref, w2_ref, b2_ref,
             s_ref, bb_ref, o_ref):
        z = h_ref[...] + a0_ref[...] + a1_ref[...]
        z = jnp.dot(z, w1_ref[...], preferred_element_type=jnp.float32,
                    precision=lax.Precision.HIGHEST) + b1_ref[...]
        z = jnp.maximum(z, 0.0)
        z = jnp.dot(z, w2_ref[...], preferred_element_type=jnp.float32,
                    precision=lax.Precision.HIGHEST) + b2_ref[...]
        z = z * s_ref[...] + bb_ref[...]
        o_ref[...] = jnp.maximum(z, 0.0)

    return pl.pallas_call(
        body,
        grid=(N // _BLK,),
        in_specs=[
            pl.BlockSpec((_BLK, H), lambda i: (i, 0)),
            pl.BlockSpec((_BLK, H), lambda i: (i, 0)),
            pl.BlockSpec((_BLK, H), lambda i: (i, 0)),
            pl.BlockSpec((H, H), lambda i: (0, 0)),
            pl.BlockSpec((1, H), lambda i: (0, 0)),
            pl.BlockSpec((H, H), lambda i: (0, 0)),
            pl.BlockSpec((1, H), lambda i: (0, 0)),
            pl.BlockSpec((1, H), lambda i: (0, 0)),
            pl.BlockSpec((1, H), lambda i: (0, 0)),
        ],
        out_specs=pl.BlockSpec((_BLK, H), lambda i: (i, 0)),
        out_shape=jax.ShapeDtypeStruct((N, H), jnp.float32),
    )(h, a0, a1, W1, b1.reshape(1, H), W2, b2.reshape(1, H),
      bn_scale, bn_b.reshape(1, H))


def _pool(h, Wp, bp):
    N, H = h.shape

    def body(h_ref, wp_ref, bp_ref, o_ref):
        hm = jnp.mean(h_ref[...], axis=0, keepdims=True)
        hx = jnp.max(h_ref[...], axis=0, keepdims=True)
        hc = jnp.concatenate([hm, hx], axis=1)
        o = jnp.dot(hc, wp_ref[...], preferred_element_type=jnp.float32,
                    precision=lax.Precision.HIGHEST) + bp_ref[...]
        o_ref[...] = jnp.maximum(o, 0.0)

    return pl.pallas_call(
        body,
        out_shape=jax.ShapeDtypeStruct((1, H), jnp.float32),
    )(h, Wp, bp.reshape(1, H))


def kernel(x, edge_index, params):
    src = edge_index[0]
    dst = edge_index[1]
    h = _input_proj(x, params["W_in"], params["b_in"],
                    params["ln_g"], params["ln_b"])
    for lp in params["layers"]:
        agg2 = _segment_sum_sc(h, src, dst)
        h = _gin_mlp(h, agg2[0], agg2[1], lp["W1"], lp["b1"],
                     lp["W2"], lp["b2"], lp["bn_g"], lp["bn_b"])
    return _pool(h, params["W_pool"], params["b_pool"])


# speedup vs baseline: 9.6935x; 1.5238x over previous
import functools

import jax
import jax.numpy as jnp
import numpy as np
from jax import lax
from jax.experimental import pallas as pl
from jax.experimental.pallas import tpu as pltpu
from jax.experimental.pallas import tpu_sc as plsc

_NC = 2
_NS = 16
_CHUNK = 80


def _segment_sum_sc(h, src, dst):
    N, H = h.shape
    E = src.shape[0]
    nw = _NC * _NS
    epw = E // nw
    nchunk = epw // _CHUNK
    assert epw * nw == E and nchunk * _CHUNK == epw
    wrows = 40
    nwc = N // wrows
    assert nwc * wrows == N

    mesh = plsc.VectorSubcoreMesh(core_axis_name="c", subcore_axis_name="s")

    R = 4
    assert (nchunk - 1) % R == 0

    scratch = (
        [pltpu.VMEM((_CHUNK,), jnp.int32) for _ in range(R)]
        + [pltpu.VMEM((_CHUNK,), jnp.int32) for _ in range(R)]
        + [pltpu.VMEM((_CHUNK, H), jnp.float32) for _ in range(R)]
        + [
            pltpu.VMEM((wrows, H), jnp.float32),
            pltpu.VMEM_SHARED((N, H), jnp.float32),
        ]
        + [pltpu.SemaphoreType.DMA for _ in range(3 * R)]
    )

    @functools.partial(
        pl.kernel,
        out_type=jax.ShapeDtypeStruct((_NC, N, H), jnp.float32),
        mesh=mesh,
        scratch_types=scratch,
    )
    def seg(h_hbm, src_hbm, dst_hbm, out_hbm, *sc):
        sidx = sc[0:R]
        didx = sc[R:2 * R]
        rows = sc[2 * R:3 * R]
        zbuf = sc[3 * R]
        agg = sc[3 * R + 1]
        isem = sc[3 * R + 2:3 * R + 2 + R]
        gsem = sc[3 * R + 2 + R:3 * R + 2 + 2 * R]
        ssem = sc[3 * R + 2 + 2 * R:3 * R + 2 + 3 * R]

        cid = lax.axis_index("c")
        sid = lax.axis_index("s")
        wid = cid * _NS + sid
        ebase = wid * epw

        def fire_idx(c, b):
            off = ebase + c * _CHUNK
            pltpu.async_copy(src_hbm.at[pl.ds(off, _CHUNK)], sidx[b], isem[b])
            pltpu.async_copy(dst_hbm.at[pl.ds(off, _CHUNK)], didx[b], isem[b])

        def wait_idx(b):
            pltpu.make_async_copy(src_hbm.at[pl.ds(0, _CHUNK)], sidx[b],
                                  isem[b]).wait()
            pltpu.make_async_copy(dst_hbm.at[pl.ds(0, _CHUNK)], didx[b],
                                  isem[b]).wait()

        def fire_gather(b):
            pltpu.async_copy(h_hbm.at[sidx[b]], rows[b], gsem[b])

        def wait_gather(b):
            pltpu.make_async_copy(h_hbm.at[pl.ds(0, _CHUNK)], rows[b],
                                  gsem[b]).wait()

        def fire_scatter(b):
            pltpu.async_copy(rows[b], agg.at[didx[b]], ssem[b], add=True)

        def wait_scatter(b):
            pltpu.make_async_copy(rows[b], agg.at[didx[b]], ssem[b]).wait()

        fire_idx(0, 0)
        fire_idx(1, 1)
        wait_idx(0)
        fire_gather(0)

        zero16 = jnp.zeros((16,), jnp.float32)

        @pl.loop(0, wrows)
        def _zero_rows(i):
            for j in range(H // 16):
                zbuf[i, pl.ds(j * 16, 16)] = zero16

        for i in range((nwc + _NS - 1) // _NS):
            blk = sid + i * _NS

            @pl.when(blk < nwc)
            def _():
                pltpu.sync_copy(zbuf, agg.at[pl.ds(blk * wrows, wrows)])
        plsc.subcore_barrier()

        @pl.loop(0, nchunk - 1, step=R)
        def _edge_block(c):
            for k in range(R):
                j = c + k
                s1 = (k + 1) % R
                s2 = (k + 2) % R

                @pl.when(j >= 2)
                def _():
                    wait_scatter(s2)

                @pl.when(j + 2 < nchunk)
                def _():
                    fire_idx(j + 2, s2)

                @pl.when(j + 1 < nchunk)
                def _():
                    wait_idx(s1)
                    fire_gather(s1)

                wait_gather(k)
                fire_scatter(k)

        last = (nchunk - 1) % R
        wait_gather(last)
        fire_scatter(last)
        wait_scatter((last + 2) % R)
        wait_scatter((last + 3) % R)
        wait_scatter(last)

        plsc.subcore_barrier()
        for i in range((nwc + _NS - 1) // _NS):
            blk = sid + i * _NS

            @pl.when(blk < nwc)
            def _():
                pltpu.sync_copy(agg.at[pl.ds(blk * wrows, wrows)],
                                out_hbm.at[cid, pl.ds(blk * wrows, wrows)])

    return seg(h, src, dst)


_BLK = 1000


def _input_proj(x, W, b, g, beta):
    N, D = x.shape
    H = W.shape[1]

    def body(x_ref, w_ref, b_ref, g_ref, bb_ref, o_ref):
        h = jnp.dot(x_ref[...], w_ref[...],
                    preferred_element_type=jnp.float32,
                    precision=lax.Precision.HIGHEST) + b_ref[...]
        h = jnp.maximum(h, 0.0)
        mu = jnp.mean(h, axis=1, keepdims=True)
        var = jnp.mean((h - mu) ** 2, axis=1, keepdims=True)
        o_ref[...] = (h - mu) * lax.rsqrt(var + 1e-5) * g_ref[...] + bb_ref[...]

    return pl.pallas_call(
        body,
        grid=(N // _BLK,),
        in_specs=[
            pl.BlockSpec((_BLK, D), lambda i: (i, 0)),
            pl.BlockSpec((D, H), lambda i: (0, 0)),
            pl.BlockSpec((1, H), lambda i: (0, 0)),
            pl.BlockSpec((1, H), lambda i: (0, 0)),
            pl.BlockSpec((1, H), lambda i: (0, 0)),
        ],
        out_specs=pl.BlockSpec((_BLK, H), lambda i: (i, 0)),
        out_shape=jax.ShapeDtypeStruct((N, H), jnp.float32),
    )(x, W, b.reshape(1, H), g.reshape(1, H), beta.reshape(1, H))


def _gin_mlp(h, a0, a1, W1, b1, W2, b2, bn_g, bn_b):
    N, H = h.shape
    bn_scale = (bn_g / np.sqrt(1.0 + 1e-5)).reshape(1, H)

    def body(h_ref, a0_ref, a1_ref, w1_ref, b1_ref, w2_ref, b2_ref,
             s_ref, bb_ref, o_ref):
        z = h_ref[...] + a0_ref[...] + a1_ref[...]
        z = jnp.dot(z, w1_ref[...], preferred_element_type=jnp.float32,
                    precision=lax.Precision.HIGHEST) + b1_ref[...]
        z = jnp.maximum(z, 0.0)
        z = jnp.dot(z, w2_ref[...], preferred_element_type=jnp.float32,
                    precision=lax.Precision.HIGHEST) + b2_ref[...]
        z = z * s_ref[...] + bb_ref[...]
        o_ref[...] = jnp.maximum(z, 0.0)

    return pl.pallas_call(
        body,
        grid=(N // _BLK,),
        in_specs=[
            pl.BlockSpec((_BLK, H), lambda i: (i, 0)),
            pl.BlockSpec((_BLK, H), lambda i: (i, 0)),
            pl.BlockSpec((_BLK, H), lambda i: (i, 0)),
            pl.BlockSpec((H, H), lambda i: (0, 0)),
            pl.BlockSpec((1, H), lambda i: (0, 0)),
            pl.BlockSpec((H, H), lambda i: (0, 0)),
            pl.BlockSpec((1, H), lambda i: (0, 0)),
            pl.BlockSpec((1, H), lambda i: (0, 0)),
            pl.BlockSpec((1, H), lambda i: (0, 0)),
        ],
        out_specs=pl.BlockSpec((_BLK, H), lambda i: (i, 0)),
        out_shape=jax.ShapeDtypeStruct((N, H), jnp.float32),
    )(h, a0, a1, W1, b1.reshape(1, H), W2, b2.reshape(1, H),
      bn_scale, bn_b.reshape(1, H))


def _pool(h, Wp, bp):
    N, H = h.shape

    def body(h_ref, wp_ref, bp_ref, o_ref):
        hm = jnp.mean(h_ref[...], axis=0, keepdims=True)
        hx = jnp.max(h_ref[...], axis=0, keepdims=True)
        hc = jnp.concatenate([hm, hx], axis=1)
        o = jnp.dot(hc, wp_ref[...], preferred_element_type=jnp.float32,
                    precision=lax.Precision.HIGHEST) + bp_ref[...]
        o_ref[...] = jnp.maximum(o, 0.0)

    return pl.pallas_call(
        body,
        out_shape=jax.ShapeDtypeStruct((1, H), jnp.float32),
    )(h, Wp, bp.reshape(1, H))


def kernel(x, edge_index, params):
    src = edge_index[0]
    dst = edge_index[1]
    h = _input_proj(x, params["W_in"], params["b_in"],
                    params["ln_g"], params["ln_b"])
    for lp in params["layers"]:
        agg2 = _segment_sum_sc(h, src, dst)
        h = _gin_mlp(h, agg2[0], agg2[1], lp["W1"], lp["b1"],
                     lp["W2"], lp["b2"], lp["bn_g"], lp["bn_b"])
    return _pool(h, params["W_pool"], params["b_pool"])


# speedup vs baseline: 12.3655x; 1.2757x over previous
import functools

import jax
import jax.numpy as jnp
import numpy as np
from jax import lax
from jax.experimental import pallas as pl
from jax.experimental.pallas import tpu as pltpu
from jax.experimental.pallas import tpu_sc as plsc

_NC = 2
_NS = 16
_CHUNK = 80


def _segment_sum_sc(h, src, dst):
    N, H = h.shape
    E = src.shape[0]
    nw = _NC * _NS
    epw = E // nw
    nchunk = epw // _CHUNK
    assert epw * nw == E and nchunk * _CHUNK == epw
    wrows = 40
    nwc = N // wrows
    assert nwc * wrows == N

    mesh = plsc.VectorSubcoreMesh(core_axis_name="c", subcore_axis_name="s")

    R = 4
    assert (nchunk - 1) % R == 0

    scratch = (
        [pltpu.VMEM((_CHUNK,), jnp.int32) for _ in range(R)]
        + [pltpu.VMEM((_CHUNK,), jnp.int32) for _ in range(R)]
        + [pltpu.VMEM((_CHUNK, H), jnp.float32) for _ in range(R)]
        + [
            pltpu.VMEM((wrows, H), jnp.float32),
            pltpu.VMEM_SHARED((N, H), jnp.float32),
        ]
        + [pltpu.SemaphoreType.DMA for _ in range(3 * R)]
    )

    @functools.partial(
        pl.kernel,
        out_type=jax.ShapeDtypeStruct((_NC, N, H), jnp.float32),
        mesh=mesh,
        scratch_types=scratch,
    )
    def seg(h_hbm, src_hbm, dst_hbm, out_hbm, *sc):
        sidx = sc[0:R]
        didx = sc[R:2 * R]
        rows = sc[2 * R:3 * R]
        zbuf = sc[3 * R]
        agg = sc[3 * R + 1]
        isem = sc[3 * R + 2:3 * R + 2 + R]
        gsem = sc[3 * R + 2 + R:3 * R + 2 + 2 * R]
        ssem = sc[3 * R + 2 + 2 * R:3 * R + 2 + 3 * R]

        cid = lax.axis_index("c")
        sid = lax.axis_index("s")
        wid = cid * _NS + sid
        ebase = wid * epw

        def fire_idx(c, b):
            off = ebase + c * _CHUNK
            pltpu.async_copy(src_hbm.at[pl.ds(off, _CHUNK)], sidx[b], isem[b])
            pltpu.async_copy(dst_hbm.at[pl.ds(off, _CHUNK)], didx[b], isem[b])

        def wait_idx(b):
            pltpu.make_async_copy(src_hbm.at[pl.ds(0, _CHUNK)], sidx[b],
                                  isem[b]).wait()
            pltpu.make_async_copy(dst_hbm.at[pl.ds(0, _CHUNK)], didx[b],
                                  isem[b]).wait()

        def fire_gather(b):
            pltpu.async_copy(h_hbm.at[sidx[b]], rows[b], gsem[b])

        def wait_gather(b):
            pltpu.make_async_copy(h_hbm.at[pl.ds(0, _CHUNK)], rows[b],
                                  gsem[b]).wait()

        def fire_scatter(b):
            pltpu.async_copy(rows[b], agg.at[didx[b]], ssem[b], add=True)

        def wait_scatter(b):
            pltpu.make_async_copy(rows[b], agg.at[didx[b]], ssem[b]).wait()

        fire_idx(0, 0)
        fire_idx(1, 1)
        wait_idx(0)
        fire_gather(0)

        zero16 = jnp.zeros((16,), jnp.float32)

        @pl.loop(0, wrows)
        def _zero_rows(i):
            for j in range(H // 16):
                zbuf[i, pl.ds(j * 16, 16)] = zero16

        for i in range((nwc + _NS - 1) // _NS):
            blk = sid + i * _NS

            @pl.when(blk < nwc)
            def _():
                pltpu.sync_copy(zbuf, agg.at[pl.ds(blk * wrows, wrows)])
        plsc.subcore_barrier()

        @pl.loop(0, nchunk - 1, step=R)
        def _edge_block(c):
            for k in range(R):
                j = c + k
                s1 = (k + 1) % R
                s2 = (k + 2) % R

                @pl.when(j >= 2)
                def _():
                    wait_scatter(s2)

                @pl.when(j + 2 < nchunk)
                def _():
                    fire_idx(j + 2, s2)

                @pl.when(j + 1 < nchunk)
                def _():
                    wait_idx(s1)
                    fire_gather(s1)

                wait_gather(k)
                fire_scatter(k)

        last = (nchunk - 1) % R
        wait_gather(last)
        fire_scatter(last)
        wait_scatter((last + 2) % R)
        wait_scatter((last + 3) % R)
        wait_scatter(last)

        plsc.subcore_barrier()
        for i in range((nwc + _NS - 1) // _NS):
            blk = sid + i * _NS

            @pl.when(blk < nwc)
            def _():
                pltpu.sync_copy(agg.at[pl.ds(blk * wrows, wrows)],
                                out_hbm.at[cid, pl.ds(blk * wrows, wrows)])

    return seg(h, src, dst)


_BLK = 2000


def _input_proj(x, W, b, g, beta):
    N, D = x.shape
    H = W.shape[1]

    def body(x_ref, w_ref, b_ref, g_ref, bb_ref, o_ref):
        h = jnp.dot(x_ref[...], w_ref[...],
                    preferred_element_type=jnp.float32) + b_ref[...]
        h = jnp.maximum(h, 0.0)
        mu = jnp.mean(h, axis=1, keepdims=True)
        var = jnp.mean((h - mu) ** 2, axis=1, keepdims=True)
        o_ref[...] = (h - mu) * lax.rsqrt(var + 1e-5) * g_ref[...] + bb_ref[...]

    return pl.pallas_call(
        body,
        grid=(N // _BLK,),
        in_specs=[
            pl.BlockSpec((_BLK, D), lambda i: (i, 0)),
            pl.BlockSpec((D, H), lambda i: (0, 0)),
            pl.BlockSpec((1, H), lambda i: (0, 0)),
            pl.BlockSpec((1, H), lambda i: (0, 0)),
            pl.BlockSpec((1, H), lambda i: (0, 0)),
        ],
        out_specs=pl.BlockSpec((_BLK, H), lambda i: (i, 0)),
        out_shape=jax.ShapeDtypeStruct((N, H), jnp.float32),
    )(x, W, b.reshape(1, H), g.reshape(1, H), beta.reshape(1, H))


def _gin_mlp(h, agg2, W1, b1, W2, b2, bn_g, bn_b):
    N, H = h.shape
    s = bn_g / np.sqrt(1.0 + 1e-5)
    W2f = W2 * s[None, :]
    b2f = b2 * s + bn_b

    def body(h_ref, a0_ref, a1_ref, w1_ref, b1_ref, w2_ref, b2_ref, o_ref):
        z = h_ref[...] + a0_ref[0] + a1_ref[0]
        z = jnp.dot(z, w1_ref[...],
                    preferred_element_type=jnp.float32) + b1_ref[...]
        z = jnp.maximum(z, 0.0)
        z = jnp.dot(z, w2_ref[...],
                    preferred_element_type=jnp.float32) + b2_ref[...]
        o_ref[...] = jnp.maximum(z, 0.0)

    return pl.pallas_call(
        body,
        grid=(N // _BLK,),
        in_specs=[
            pl.BlockSpec((_BLK, H), lambda i: (i, 0)),
            pl.BlockSpec((1, _BLK, H), lambda i: (0, i, 0)),
            pl.BlockSpec((1, _BLK, H), lambda i: (1, i, 0)),
            pl.BlockSpec((H, H), lambda i: (0, 0)),
            pl.BlockSpec((1, H), lambda i: (0, 0)),
            pl.BlockSpec((H, H), lambda i: (0, 0)),
            pl.BlockSpec((1, H), lambda i: (0, 0)),
        ],
        out_specs=pl.BlockSpec((_BLK, H), lambda i: (i, 0)),
        out_shape=jax.ShapeDtypeStruct((N, H), jnp.float32),
    )(h, agg2, agg2, W1, b1.reshape(1, H), W2f, b2f.reshape(1, H))


def _pool(h, Wp, bp):
    N, H = h.shape

    def body(h_ref, wp_ref, bp_ref, o_ref):
        hm = jnp.mean(h_ref[...], axis=0, keepdims=True)
        hx = jnp.max(h_ref[...], axis=0, keepdims=True)
        hc = jnp.concatenate([hm, hx], axis=1)
        o = jnp.dot(hc, wp_ref[...],
                    preferred_element_type=jnp.float32) + bp_ref[...]
        o_ref[...] = jnp.maximum(o, 0.0)

    return pl.pallas_call(
        body,
        out_shape=jax.ShapeDtypeStruct((1, H), jnp.float32),
    )(h, Wp, bp.reshape(1, H))


def kernel(x, edge_index, params):
    src = edge_index[0]
    dst = edge_index[1]
    h = _input_proj(x, params["W_in"], params["b_in"],
                    params["ln_g"], params["ln_b"])
    for lp in params["layers"]:
        agg2 = _segment_sum_sc(h, src, dst)
        h = _gin_mlp(h, agg2, lp["W1"], lp["b1"],
                     lp["W2"], lp["b2"], lp["bn_g"], lp["bn_b"])
    return _pool(h, params["W_pool"], params["b_pool"])
